# baseline (device time: 74209 ns/iter reference)
import jax
import jax.numpy as jnp
from jax import lax
from jax.experimental import pallas as pl
from jax.experimental.pallas import tpu as pltpu

N_DEV = 4
SQ = 512
D = 1024
H = 8
DH = 128
SCALE = 0.08838834764831843


def _body(x_ref, wq_ref, wo_ref, wk_ref, wv_ref, out_ref,
          send_ref, recv_ref, send_sems, recv_sems):
    my = lax.axis_index("i")

    xv = x_ref[...].astype(jnp.bfloat16)
    wq = wq_ref[...].astype(jnp.bfloat16)
    wk = wk_ref[...].astype(jnp.bfloat16)
    wv = wv_ref[...].astype(jnp.bfloat16)
    wo = wo_ref[...].astype(jnp.bfloat16)

    dn = (((1,), (0,)), ((), ()))
    q_all = lax.dot_general(xv, wq, dn, preferred_element_type=jnp.float32)
    k_all = lax.dot_general(xv, wk, dn, preferred_element_type=jnp.float32)
    v_all = lax.dot_general(xv, wv, dn, preferred_element_type=jnp.float32)
    q_all = q_all.astype(jnp.bfloat16)
    k_all = k_all.astype(jnp.bfloat16)
    v_all = v_all.astype(jnp.bfloat16)

    outs = []
    for h in range(H):
        sl = slice(h * DH, (h + 1) * DH)
        q = q_all[:, sl]
        k = k_all[:, sl]
        v = v_all[:, sl]
        s = lax.dot_general(q, k, (((1,), (1,)), ((), ())),
                            preferred_element_type=jnp.float32) * SCALE
        m = jnp.max(s, axis=1, keepdims=True)
        p = jnp.exp(s - m)
        l = jnp.sum(p, axis=1, keepdims=True)
        o = lax.dot_general(p.astype(jnp.bfloat16), v, dn,
                            preferred_element_type=jnp.float32)
        outs.append((o / l).astype(jnp.bfloat16))
    attn = jnp.concatenate(outs, axis=1)

    partial = lax.dot_general(attn, wo, dn, preferred_element_type=jnp.float32)

    send_ref[...] = partial
    p1 = my ^ 1
    rdma1 = pltpu.make_async_remote_copy(
        src_ref=send_ref,
        dst_ref=recv_ref.at[0],
        send_sem=send_sems.at[0],
        recv_sem=recv_sems.at[0],
        device_id=(p1,),
        device_id_type=pl.DeviceIdType.MESH,
    )
    rdma1.start()
    rdma1.wait()

    acc = partial + recv_ref[0]
    send_ref[...] = acc
    p2 = 3 - my
    rdma2 = pltpu.make_async_remote_copy(
        src_ref=send_ref,
        dst_ref=recv_ref.at[1],
        send_sem=send_sems.at[1],
        recv_sem=recv_sems.at[1],
        device_id=(p2,),
        device_id_type=pl.DeviceIdType.MESH,
    )
    rdma2.start()
    rdma2.wait()

    out_ref[...] = acc + recv_ref[1]


def kernel(x, Wq, Wo, Wk, Wv):
    x2 = x.reshape(SQ, D)
    out = pl.pallas_call(
        _body,
        out_shape=jax.ShapeDtypeStruct((SQ, D), jnp.float32),
        in_specs=[pl.BlockSpec(memory_space=pltpu.VMEM)] * 5,
        out_specs=pl.BlockSpec(memory_space=pltpu.VMEM),
        scratch_shapes=[
            pltpu.VMEM((SQ, D), jnp.float32),
            pltpu.VMEM((2, SQ, D), jnp.float32),
            pltpu.SemaphoreType.DMA((2,)),
            pltpu.SemaphoreType.DMA((2,)),
        ],
    )(x2, Wq, Wo, Wk, Wv)
    return out.reshape(1, SQ, D)


# device time: 19793 ns/iter; 3.7493x vs baseline; 3.7493x over previous
import jax
import jax.numpy as jnp
from jax import lax
from jax.experimental import pallas as pl
from jax.experimental.pallas import tpu as pltpu

N_DEV = 4
SQ = 512
D = 1024
H = 8
DH = 128
SCALE = 0.08838834764831843


def _body(x_ref, wq_ref, wo_ref, wk_ref, wv_ref, out_ref,
          send_ref, recv_ref, send_sems, recv_sems):
    my = lax.axis_index("i")

    xv = x_ref[...].astype(jnp.bfloat16)
    wq = wq_ref[...].astype(jnp.bfloat16)
    wk = wk_ref[...].astype(jnp.bfloat16)
    wv = wv_ref[...].astype(jnp.bfloat16)
    wo = wo_ref[...].astype(jnp.bfloat16)

    dn = (((1,), (0,)), ((), ()))
    q_all = lax.dot_general(xv, wq, dn, preferred_element_type=jnp.float32)
    k_all = lax.dot_general(xv, wk, dn, preferred_element_type=jnp.float32)
    v_all = lax.dot_general(xv, wv, dn, preferred_element_type=jnp.float32)
    q_all = q_all.astype(jnp.bfloat16)
    k_all = k_all.astype(jnp.bfloat16)
    v_all = v_all.astype(jnp.bfloat16)

    outs = []
    for h in range(H):
        sl = slice(h * DH, (h + 1) * DH)
        q = q_all[:, sl]
        k = k_all[:, sl]
        v = v_all[:, sl]
        s = lax.dot_general(q, k, (((1,), (1,)), ((), ())),
                            preferred_element_type=jnp.float32) * SCALE
        m = jnp.max(s, axis=1, keepdims=True)
        p = jnp.exp(s - m)
        l = jnp.sum(p, axis=1, keepdims=True)
        o = lax.dot_general(p.astype(jnp.bfloat16), v, dn,
                            preferred_element_type=jnp.float32)
        outs.append((o / l).astype(jnp.bfloat16))
    attn = jnp.concatenate(outs, axis=1)

    partial = lax.dot_general(attn, wo, dn, preferred_element_type=jnp.float32)

    p1 = my ^ 1
    p2 = 3 - my
    HW = D // 2

    def exchange(stage, vals, partners):
        rdmas = []
        for half in range(2):
            send_ref[stage, half] = vals[half].astype(jnp.bfloat16)
            rdma = pltpu.make_async_remote_copy(
                src_ref=send_ref.at[stage, half],
                dst_ref=recv_ref.at[stage, half],
                send_sem=send_sems.at[stage, half],
                recv_sem=recv_sems.at[stage, half],
                device_id=(partners[half],),
                device_id_type=pl.DeviceIdType.MESH,
            )
            rdma.start()
            rdmas.append(rdma)
        out = []
        for half in range(2):
            rdmas[half].wait()
            out.append(vals[half] + recv_ref[stage, half].astype(jnp.float32))
        return out

    halves = [partial[:, :HW], partial[:, HW:]]
    halves = exchange(0, halves, (p1, p2))
    halves = exchange(1, halves, (p2, p1))
    out_ref[:, :HW] = halves[0]
    out_ref[:, HW:] = halves[1]


def kernel(x, Wq, Wo, Wk, Wv):
    x2 = x.reshape(SQ, D)
    out = pl.pallas_call(
        _body,
        out_shape=jax.ShapeDtypeStruct((SQ, D), jnp.float32),
        in_specs=[pl.BlockSpec(memory_space=pltpu.VMEM)] * 5,
        out_specs=pl.BlockSpec(memory_space=pltpu.VMEM),
        scratch_shapes=[
            pltpu.VMEM((2, 2, SQ, D // 2), jnp.bfloat16),
            pltpu.VMEM((2, 2, SQ, D // 2), jnp.bfloat16),
            pltpu.SemaphoreType.DMA((2, 2)),
            pltpu.SemaphoreType.DMA((2, 2)),
        ],
    )(x2, Wq, Wo, Wk, Wv)
    return out.reshape(1, SQ, D)
